# baseline (speedup 1.0000x reference)
import functools

import jax
import jax.numpy as jnp
from jax import lax
from jax.experimental import pallas as pl
from jax.experimental.pallas import tpu as pltpu
from jax.experimental.pallas import tpu_sc as plsc

NC = 2
NS = 16
LANES = 16
CH = 128
FW = 128
RNB = 2
INB = 4


def _sc_apply_fn(n, ept):
    nch = ept // CH
    assert nch % INB == 0 and nch >= 2 * INB
    ngrp = nch // INB
    npt = n // NS
    mesh = plsc.VectorSubcoreMesh(core_axis_name="c", subcore_axis_name="s")

    @functools.partial(
        pl.kernel,
        out_type=jax.ShapeDtypeStruct((NC, n, FW), jnp.float32),
        mesh=mesh,
        compiler_params=pltpu.CompilerParams(needs_layout_passes=False),
        scratch_types=[
            pltpu.VMEM((INB * 3, CH), jnp.int32),
            pltpu.VMEM((RNB, CH, FW), jnp.float32),
            pltpu.VMEM_SHARED((n, FW), jnp.float32),
            pltpu.SemaphoreType.DMA((INB,)),
            pltpu.SemaphoreType.DMA((RNB,)),
            pltpu.SemaphoreType.DMA((RNB,)),
        ],
    )
    def k(table_h, edg_h, zeros_h, out_h,
          idx_v, rows_v, agg_s, sem_i, sem_g, sem_s):
        c = lax.axis_index("c")
        s = lax.axis_index("s")
        pltpu.sync_copy(zeros_h.at[pl.ds(s * npt, npt)],
                        agg_s.at[pl.ds(s * npt, npt)])
        plsc.subcore_barrier()

        def idx_issue(j, q):
            pltpu.async_copy(edg_h.at[c, s, j],
                             idx_v.at[pl.ds(q * 3, 3)], sem_i.at[q])

        def idx_wait(j, q):
            pltpu.make_async_copy(edg_h.at[c, s, j],
                                  idx_v.at[pl.ds(q * 3, 3)], sem_i.at[q]).wait()

        def gather_issue(q, rb):
            pltpu.async_copy(
                table_h.at[idx_v.at[q * 3]], rows_v.at[rb], sem_g.at[rb])

        def gather_wait(q, rb):
            pltpu.make_async_copy(
                table_h.at[idx_v.at[q * 3]], rows_v.at[rb], sem_g.at[rb]).wait()

        def scatter_issue(q, rb):
            pltpu.async_copy(rows_v.at[rb], agg_s.at[idx_v.at[q * 3 + 1]],
                             sem_s.at[rb], add=True)

        def scatter_wait(q, rb):
            pltpu.make_async_copy(rows_v.at[rb], agg_s.at[idx_v.at[q * 3 + 1]],
                                  sem_s.at[rb]).wait()

        def scale(q, rb):
            def edge(j2, carry):
                wbits = plsc.load_gather(
                    idx_v, [jnp.full((LANES,), q * 3 + 2, jnp.int32),
                            jnp.full((LANES,), j2, jnp.int32)])
                w = plsc.bitcast(wbits, jnp.float32)
                for kk in range(FW // LANES):
                    sl = pl.ds(kk * LANES, LANES)
                    rows_v[rb, j2, sl] = rows_v[rb, j2, sl] * w
                return carry

            lax.fori_loop(0, CH, edge, 0, unroll=2)

        def chunk_body(j, u, do_sw=True, do_i2=True, do_g1=True):
            rb, qb = u % RNB, u % INB
            if do_sw:
                scatter_wait((u - 1) % INB, (u - 1) % RNB)
            if do_i2:
                idx_issue(j + 2, (u + 2) % INB)
            if do_g1:
                idx_wait(j + 1, (u + 1) % INB)
                gather_issue((u + 1) % INB, (u + 1) % RNB)
            gather_wait(qb, rb)
            scale(qb, rb)
            scatter_issue(qb, rb)

        idx_issue(0, 0)
        idx_issue(1, 1)
        idx_wait(0, 0)
        gather_issue(0, 0)
        for u in range(INB):
            chunk_body(u, u, do_sw=(u >= 1))

        def group(g, carry):
            j0 = g * INB
            for u in range(INB):
                chunk_body(j0 + u, u)
            return carry

        lax.fori_loop(1, ngrp - 1, group, 0)

        j0 = (ngrp - 1) * INB
        for u in range(INB):
            chunk_body(j0 + u, u, do_i2=(u < INB - 2), do_g1=(u < INB - 1))

        scatter_wait((INB - 1) % INB, (INB - 1) % RNB)
        plsc.subcore_barrier()
        pltpu.sync_copy(agg_s.at[pl.ds(s * npt, npt)],
                        out_h.at[c, pl.ds(s * npt, npt)])

    return k


def _tc_gate1(xg, ah, h, W1, b1, blk):
    n = xg.shape[0]
    grid = (n // blk,)

    def body(xg_ref, ah_ref, h_ref, w_ref, b_ref, rh_ref, u_ref):
        cat = jnp.concatenate([xg_ref[...], ah_ref[0], ah_ref[1]], axis=1)
        ru = jax.nn.sigmoid(
            jnp.dot(cat, w_ref[...], preferred_element_type=jnp.float32)
            + b_ref[...])
        rh_ref[0] = ru[:, :FW] * h_ref[0]
        rh_ref[1] = ru[:, FW:2 * FW] * h_ref[1]
        u_ref[...] = ru[:, 2 * FW:]

    return pl.pallas_call(
        body,
        grid=grid,
        in_specs=[
            pl.BlockSpec((blk, FW), lambda i: (i, 0)),
            pl.BlockSpec((NC, blk, FW), lambda i: (0, i, 0)),
            pl.BlockSpec((NC, blk, FW), lambda i: (0, i, 0)),
            pl.BlockSpec(W1.shape, lambda i: (0, 0)),
            pl.BlockSpec(b1.shape, lambda i: (0, 0)),
        ],
        out_specs=[
            pl.BlockSpec((NC, blk, FW), lambda i: (0, i, 0)),
            pl.BlockSpec((blk, 2 * FW), lambda i: (i, 0)),
        ],
        out_shape=[
            jax.ShapeDtypeStruct((NC, n, FW), jnp.float32),
            jax.ShapeDtypeStruct((n, 2 * FW), jnp.float32),
        ],
    )(xg, ah, h, W1, b1)


def _tc_gate2(xg, arh, u, h, W2, b2, blk):
    n = xg.shape[0]
    grid = (n // blk,)

    def body(xg_ref, arh_ref, u_ref, h_ref, w_ref, b_ref, hn_ref):
        cat = jnp.concatenate([xg_ref[...], arh_ref[0], arh_ref[1]], axis=1)
        cc = jnp.tanh(
            jnp.dot(cat, w_ref[...], preferred_element_type=jnp.float32)
            + b_ref[...])
        uu = u_ref[...]
        hcat = jnp.concatenate([h_ref[0], h_ref[1]], axis=1)
        hn = uu * hcat + (1.0 - uu) * cc
        hn_ref[0] = hn[:, :FW]
        hn_ref[1] = hn[:, FW:]

    return pl.pallas_call(
        body,
        grid=grid,
        in_specs=[
            pl.BlockSpec((blk, FW), lambda i: (i, 0)),
            pl.BlockSpec((NC, blk, FW), lambda i: (0, i, 0)),
            pl.BlockSpec((blk, 2 * FW), lambda i: (i, 0)),
            pl.BlockSpec((NC, blk, FW), lambda i: (0, i, 0)),
            pl.BlockSpec(W2.shape, lambda i: (0, 0)),
            pl.BlockSpec(b2.shape, lambda i: (0, 0)),
        ],
        out_specs=[pl.BlockSpec((NC, blk, FW), lambda i: (0, i, 0))],
        out_shape=[jax.ShapeDtypeStruct((NC, n, FW), jnp.float32)],
    )(xg, arh, u, h, W2, b2)[0]


def _tc_out(h, Wout, bout, blk):
    n = h.shape[1]
    grid = (n // blk,)

    def body(h_ref, w_ref, b_ref, o_ref):
        hcat = jnp.concatenate([h_ref[0], h_ref[1]], axis=1)
        o_ref[...] = (
            jnp.dot(hcat, w_ref[...], preferred_element_type=jnp.float32)
            + b_ref[...])

    return pl.pallas_call(
        body,
        grid=grid,
        in_specs=[
            pl.BlockSpec((NC, blk, FW), lambda i: (0, i, 0)),
            pl.BlockSpec(Wout.shape, lambda i: (0, 0)),
            pl.BlockSpec(bout.shape, lambda i: (0, 0)),
        ],
        out_specs=[pl.BlockSpec((blk, Wout.shape[1]), lambda i: (i, 0))],
        out_shape=[jax.ShapeDtypeStruct((n, Wout.shape[1]), jnp.float32)],
    )(h, Wout, bout)[0]


def kernel(x, edge_index, edge_weight, W1, b1, W2, b2, Wout, bout):
    n, f, t_steps = x.shape
    e = edge_weight.shape[0]
    hdim = W2.shape[1]
    assert f == FW and hdim == 2 * FW

    npad = -(-n // (8 * NS)) * (8 * NS)

    nch_min = -(-e // (NS * CH))
    nch_tot = max(2 * INB, -(-nch_min // INB) * INB)
    epad = NS * CH * nch_tot
    src = jnp.pad(edge_index[0], (0, epad - e))
    dst = jnp.pad(edge_index[1], (0, epad - e))
    ew = jnp.pad(edge_weight, (0, epad - e))
    ept = epad // NS
    nch = ept // CH
    src_cs = src.reshape(NS, nch, CH)
    src_t = jnp.stack([src_cs, src_cs + npad], axis=0)
    dst_t = jnp.broadcast_to(dst.reshape(NS, nch, CH), (NC, NS, nch, CH))
    ew_bits = lax.bitcast_convert_type(ew, jnp.int32)
    ew_t = jnp.broadcast_to(ew_bits.reshape(NS, nch, CH), (NC, NS, nch, CH))
    edg_t = jnp.stack([src_t, dst_t, ew_t], axis=3)
    zeros_nf = jnp.zeros((npad, FW), jnp.float32)

    sc_apply = _sc_apply_fn(npad, ept)

    def apply_a(table):
        return sc_apply(table, edg_t, zeros_nf)

    x_t = jnp.transpose(x, (2, 0, 1))
    x_t = jnp.pad(x_t, ((0, 0), (0, npad - n), (0, 0)))
    x_pairs = x_t.reshape(t_steps // 2, 2 * npad, f)
    xg = [apply_a(x_pairs[k]) for k in range(t_steps // 2)]

    b1r = b1.reshape(1, -1)
    b2r = b2.reshape(1, -1)
    boutr = bout.reshape(1, -1)
    blk = next(b for b in range(2024, 0, -8) if npad % b == 0)

    h = jnp.zeros((NC, npad, FW), jnp.float32)
    ah = jnp.zeros((NC, npad, FW), jnp.float32)
    for t in range(t_steps):
        xg_t = xg[t // 2][t % 2]
        if t > 0:
            ah = apply_a(h.reshape(2 * npad, FW))
        rh, u = _tc_gate1(xg_t, ah, h, W1, b1r, blk)
        arh = apply_a(rh.reshape(2 * npad, FW))
        h = _tc_gate2(xg_t, arh, u, h, W2, b2r, blk)
    return _tc_out(h, Wout, boutr, blk)[:n]


# speedup vs baseline: 1.0625x; 1.0625x over previous
import functools

import jax
import jax.numpy as jnp
from jax import lax
from jax.experimental import pallas as pl
from jax.experimental.pallas import tpu as pltpu
from jax.experimental.pallas import tpu_sc as plsc

NC = 2
NS = 16
LANES = 16
CH = 128
FW = 128
RNB = 2
INB = 4


def _sc_apply_fn(n, ept):
    nch = ept // CH
    assert nch % INB == 0 and nch >= 2 * INB
    ngrp = nch // INB
    npt = n // NS
    mesh = plsc.VectorSubcoreMesh(core_axis_name="c", subcore_axis_name="s")

    @functools.partial(
        pl.kernel,
        out_type=jax.ShapeDtypeStruct((NC, n, FW), jnp.float32),
        mesh=mesh,
        compiler_params=pltpu.CompilerParams(needs_layout_passes=False),
        scratch_types=[
            pltpu.VMEM((INB * 3, CH), jnp.int32),
            pltpu.VMEM((RNB, CH, FW), jnp.float32),
            pltpu.VMEM_SHARED((n, FW), jnp.float32),
            pltpu.SemaphoreType.DMA((INB,)),
            pltpu.SemaphoreType.DMA((RNB,)),
            pltpu.SemaphoreType.DMA((RNB,)),
        ],
    )
    def k(table_h, edg_h, zeros_h, out_h,
          idx_v, rows_v, agg_s, sem_i, sem_g, sem_s):
        c = lax.axis_index("c")
        s = lax.axis_index("s")
        pltpu.sync_copy(zeros_h.at[pl.ds(s * npt, npt)],
                        agg_s.at[pl.ds(s * npt, npt)])
        plsc.subcore_barrier()

        def idx_issue(j, q):
            pltpu.async_copy(edg_h.at[c, s, j],
                             idx_v.at[pl.ds(q * 3, 3)], sem_i.at[q])

        def idx_wait(j, q):
            pltpu.make_async_copy(edg_h.at[c, s, j],
                                  idx_v.at[pl.ds(q * 3, 3)], sem_i.at[q]).wait()

        def gather_issue(q, rb):
            pltpu.async_copy(
                table_h.at[idx_v.at[q * 3]], rows_v.at[rb], sem_g.at[rb])

        def gather_wait(q, rb):
            pltpu.make_async_copy(
                table_h.at[idx_v.at[q * 3]], rows_v.at[rb], sem_g.at[rb]).wait()

        def scatter_issue(q, rb):
            pltpu.async_copy(rows_v.at[rb], agg_s.at[idx_v.at[q * 3 + 1]],
                             sem_s.at[rb], add=True)

        def scatter_wait(q, rb):
            pltpu.make_async_copy(rows_v.at[rb], agg_s.at[idx_v.at[q * 3 + 1]],
                                  sem_s.at[rb]).wait()

        def scale(q, rb):
            def grp(g, carry):
                g0 = g * LANES
                wv = plsc.bitcast(idx_v[q * 3 + 2, pl.ds(g0, LANES)],
                                  jnp.float32)
                for i in range(LANES):
                    w = jnp.take_along_axis(
                        wv, jnp.full((LANES,), i, jnp.int32), axis=0)
                    for kk in range(FW // LANES):
                        sl = pl.ds(kk * LANES, LANES)
                        rows_v[rb, g0 + i, sl] = rows_v[rb, g0 + i, sl] * w
                return carry

            lax.fori_loop(0, CH // LANES, grp, 0)

        def chunk_body(j, u, do_sw=True, do_i2=True, do_g1=True):
            rb, qb = u % RNB, u % INB
            if do_sw:
                scatter_wait((u - 1) % INB, (u - 1) % RNB)
            if do_i2:
                idx_issue(j + 2, (u + 2) % INB)
            if do_g1:
                idx_wait(j + 1, (u + 1) % INB)
                gather_issue((u + 1) % INB, (u + 1) % RNB)
            gather_wait(qb, rb)
            scale(qb, rb)
            scatter_issue(qb, rb)

        idx_issue(0, 0)
        idx_issue(1, 1)
        idx_wait(0, 0)
        gather_issue(0, 0)
        for u in range(INB):
            chunk_body(u, u, do_sw=(u >= 1))

        def group(g, carry):
            j0 = g * INB
            for u in range(INB):
                chunk_body(j0 + u, u)
            return carry

        lax.fori_loop(1, ngrp - 1, group, 0)

        j0 = (ngrp - 1) * INB
        for u in range(INB):
            chunk_body(j0 + u, u, do_i2=(u < INB - 2), do_g1=(u < INB - 1))

        scatter_wait((INB - 1) % INB, (INB - 1) % RNB)
        plsc.subcore_barrier()
        pltpu.sync_copy(agg_s.at[pl.ds(s * npt, npt)],
                        out_h.at[c, pl.ds(s * npt, npt)])

    return k


def _tc_gate1(xg, ah, h, W1, b1, blk):
    n = xg.shape[0]
    grid = (n // blk,)

    def body(xg_ref, ah_ref, h_ref, w_ref, b_ref, rh_ref, u_ref):
        cat = jnp.concatenate([xg_ref[...], ah_ref[0], ah_ref[1]], axis=1)
        ru = jax.nn.sigmoid(
            jnp.dot(cat, w_ref[...], preferred_element_type=jnp.float32)
            + b_ref[...])
        rh_ref[0] = ru[:, :FW] * h_ref[0]
        rh_ref[1] = ru[:, FW:2 * FW] * h_ref[1]
        u_ref[...] = ru[:, 2 * FW:]

    return pl.pallas_call(
        body,
        grid=grid,
        in_specs=[
            pl.BlockSpec((blk, FW), lambda i: (i, 0)),
            pl.BlockSpec((NC, blk, FW), lambda i: (0, i, 0)),
            pl.BlockSpec((NC, blk, FW), lambda i: (0, i, 0)),
            pl.BlockSpec(W1.shape, lambda i: (0, 0)),
            pl.BlockSpec(b1.shape, lambda i: (0, 0)),
        ],
        out_specs=[
            pl.BlockSpec((NC, blk, FW), lambda i: (0, i, 0)),
            pl.BlockSpec((blk, 2 * FW), lambda i: (i, 0)),
        ],
        out_shape=[
            jax.ShapeDtypeStruct((NC, n, FW), jnp.float32),
            jax.ShapeDtypeStruct((n, 2 * FW), jnp.float32),
        ],
    )(xg, ah, h, W1, b1)


def _tc_gate2(xg, arh, u, h, W2, b2, blk):
    n = xg.shape[0]
    grid = (n // blk,)

    def body(xg_ref, arh_ref, u_ref, h_ref, w_ref, b_ref, hn_ref):
        cat = jnp.concatenate([xg_ref[...], arh_ref[0], arh_ref[1]], axis=1)
        cc = jnp.tanh(
            jnp.dot(cat, w_ref[...], preferred_element_type=jnp.float32)
            + b_ref[...])
        uu = u_ref[...]
        hcat = jnp.concatenate([h_ref[0], h_ref[1]], axis=1)
        hn = uu * hcat + (1.0 - uu) * cc
        hn_ref[0] = hn[:, :FW]
        hn_ref[1] = hn[:, FW:]

    return pl.pallas_call(
        body,
        grid=grid,
        in_specs=[
            pl.BlockSpec((blk, FW), lambda i: (i, 0)),
            pl.BlockSpec((NC, blk, FW), lambda i: (0, i, 0)),
            pl.BlockSpec((blk, 2 * FW), lambda i: (i, 0)),
            pl.BlockSpec((NC, blk, FW), lambda i: (0, i, 0)),
            pl.BlockSpec(W2.shape, lambda i: (0, 0)),
            pl.BlockSpec(b2.shape, lambda i: (0, 0)),
        ],
        out_specs=[pl.BlockSpec((NC, blk, FW), lambda i: (0, i, 0))],
        out_shape=[jax.ShapeDtypeStruct((NC, n, FW), jnp.float32)],
    )(xg, arh, u, h, W2, b2)[0]


def _tc_out(h, Wout, bout, blk):
    n = h.shape[1]
    grid = (n // blk,)

    def body(h_ref, w_ref, b_ref, o_ref):
        hcat = jnp.concatenate([h_ref[0], h_ref[1]], axis=1)
        o_ref[...] = (
            jnp.dot(hcat, w_ref[...], preferred_element_type=jnp.float32)
            + b_ref[...])

    return pl.pallas_call(
        body,
        grid=grid,
        in_specs=[
            pl.BlockSpec((NC, blk, FW), lambda i: (0, i, 0)),
            pl.BlockSpec(Wout.shape, lambda i: (0, 0)),
            pl.BlockSpec(bout.shape, lambda i: (0, 0)),
        ],
        out_specs=[pl.BlockSpec((blk, Wout.shape[1]), lambda i: (i, 0))],
        out_shape=[jax.ShapeDtypeStruct((n, Wout.shape[1]), jnp.float32)],
    )(h, Wout, bout)[0]


def kernel(x, edge_index, edge_weight, W1, b1, W2, b2, Wout, bout):
    n, f, t_steps = x.shape
    e = edge_weight.shape[0]
    hdim = W2.shape[1]
    assert f == FW and hdim == 2 * FW

    npad = -(-n // (8 * NS)) * (8 * NS)

    nch_min = -(-e // (NS * CH))
    nch_tot = max(2 * INB, -(-nch_min // INB) * INB)
    epad = NS * CH * nch_tot
    src = jnp.pad(edge_index[0], (0, epad - e))
    dst = jnp.pad(edge_index[1], (0, epad - e))
    ew = jnp.pad(edge_weight, (0, epad - e))
    ept = epad // NS
    nch = ept // CH
    src_cs = src.reshape(NS, nch, CH)
    src_t = jnp.stack([src_cs, src_cs + npad], axis=0)
    dst_t = jnp.broadcast_to(dst.reshape(NS, nch, CH), (NC, NS, nch, CH))
    ew_bits = lax.bitcast_convert_type(ew, jnp.int32)
    ew_t = jnp.broadcast_to(ew_bits.reshape(NS, nch, CH), (NC, NS, nch, CH))
    edg_t = jnp.stack([src_t, dst_t, ew_t], axis=3)
    zeros_nf = jnp.zeros((npad, FW), jnp.float32)

    sc_apply = _sc_apply_fn(npad, ept)

    def apply_a(table):
        return sc_apply(table, edg_t, zeros_nf)

    x_t = jnp.transpose(x, (2, 0, 1))
    x_t = jnp.pad(x_t, ((0, 0), (0, npad - n), (0, 0)))
    x_pairs = x_t.reshape(t_steps // 2, 2 * npad, f)
    xg = [apply_a(x_pairs[k]) for k in range(t_steps // 2)]

    b1r = b1.reshape(1, -1)
    b2r = b2.reshape(1, -1)
    boutr = bout.reshape(1, -1)
    blk = next(b for b in range(2024, 0, -8) if npad % b == 0)

    h = jnp.zeros((NC, npad, FW), jnp.float32)
    ah = jnp.zeros((NC, npad, FW), jnp.float32)
    for t in range(t_steps):
        xg_t = xg[t // 2][t % 2]
        if t > 0:
            ah = apply_a(h.reshape(2 * npad, FW))
        rh, u = _tc_gate1(xg_t, ah, h, W1, b1r, blk)
        arh = apply_a(rh.reshape(2 * npad, FW))
        h = _tc_gate2(xg_t, arh, u, h, W2, b2r, blk)
    return _tc_out(h, Wout, boutr, blk)[:n]
